# baseline (speedup 1.0000x reference)
import functools

import jax
import jax.numpy as jnp
from jax import lax
from jax.experimental import pallas as pl
from jax.experimental.pallas import tpu as pltpu
from jax.experimental.pallas import tpu_sc as plsc

N = 10000
E = 160000
IN_DIM = 128
HID = 16

CHW = 128
E_PAD = 163840
ROWS_PER_TILE = 40
PER_TILE = ROWS_PER_TILE * CHW
N_PER_TILE = N // 16

B_EDGE = 4096
N_BLK = E_PAD // B_EDGE


def _sc_gather_body(x_hbm, src_hbm, out_hbm, idx_v, rows_v, sem):
    wid = lax.axis_index("s") * 2 + lax.axis_index("c")
    pltpu.sync_copy(src_hbm.at[pl.ds(wid * ROWS_PER_TILE, ROWS_PER_TILE)], idx_v)

    def chunk(i, carry):
        cps = []
        for j in range(8):
            k = i * 8 + j
            cps.append(pltpu.async_copy(
                x_hbm.at[idx_v.at[k]], rows_v.at[pl.ds(k * CHW, CHW)], sem))
        for cp in cps:
            cp.wait()
        return carry

    lax.fori_loop(0, ROWS_PER_TILE // 8, chunk, 0)
    pltpu.sync_copy(rows_v, out_hbm.at[pl.ds(wid * PER_TILE, PER_TILE)])


def _sc_scatter_body(vals_hbm, dst_hbm, zeros_hbm, out_hbm, idx_v, rows_v, agg_sh,
                     sem):
    cid = lax.axis_index("c")
    sid = lax.axis_index("s")
    wid = sid * 2 + cid
    nbase = sid * N_PER_TILE
    pltpu.sync_copy(zeros_hbm.at[pl.ds(nbase, N_PER_TILE)],
                    agg_sh.at[pl.ds(nbase, N_PER_TILE)])
    pltpu.sync_copy(dst_hbm.at[pl.ds(wid * ROWS_PER_TILE, ROWS_PER_TILE)], idx_v)
    pltpu.sync_copy(vals_hbm.at[pl.ds(wid * PER_TILE, PER_TILE)], rows_v)
    plsc.subcore_barrier()

    def chunk(i, carry):
        cps = []
        for j in range(8):
            k = i * 8 + j
            cps.append(pltpu.async_copy(rows_v.at[pl.ds(k * CHW, CHW)],
                                        agg_sh.at[idx_v.at[k]], sem, add=True))
        for cp in cps:
            cp.wait()
        return carry

    lax.fori_loop(0, ROWS_PER_TILE // 8, chunk, 0)
    plsc.subcore_barrier()
    pltpu.sync_copy(agg_sh.at[pl.ds(nbase, N_PER_TILE)],
                    out_hbm.at[cid, pl.ds(nbase, N_PER_TILE)])


def _sc_count_body(ones_hbm, dst_hbm, zeros_hbm, out_hbm, idx_v, rows_v, agg_sh,
                   sem):
    cid = lax.axis_index("c")
    sid = lax.axis_index("s")
    wid = sid * 2 + cid
    nbase = sid * N_PER_TILE
    pltpu.sync_copy(zeros_hbm.at[pl.ds(nbase, N_PER_TILE)],
                    agg_sh.at[pl.ds(nbase, N_PER_TILE)])
    pltpu.sync_copy(dst_hbm.at[pl.ds(wid * ROWS_PER_TILE, ROWS_PER_TILE)], idx_v)
    pltpu.sync_copy(ones_hbm, rows_v)
    plsc.subcore_barrier()

    def chunk(i, carry):
        cps = []
        for j in range(8):
            k = i * 8 + j
            cps.append(pltpu.async_copy(rows_v.at[pl.ds(k * CHW, CHW)],
                                        agg_sh.at[idx_v.at[k]], sem, add=True))
        for cp in cps:
            cp.wait()
        return carry

    lax.fori_loop(0, ROWS_PER_TILE // 8, chunk, 0)
    plsc.subcore_barrier()
    pltpu.sync_copy(agg_sh.at[pl.ds(nbase, N_PER_TILE)],
                    out_hbm.at[cid, pl.ds(nbase, N_PER_TILE)])


@functools.cache
def _sc_kernels():
    mesh = plsc.VectorSubcoreMesh(core_axis_name="c", subcore_axis_name="s")
    sc_gather = pl.kernel(
        _sc_gather_body,
        out_type=jax.ShapeDtypeStruct((E_PAD, HID), jnp.float32),
        mesh=mesh,
        compiler_params=pltpu.CompilerParams(use_tc_tiling_on_sc=False),
        scratch_types=[
            pltpu.VMEM((ROWS_PER_TILE, CHW), jnp.int32),
            pltpu.VMEM((PER_TILE, HID), jnp.float32),
            pltpu.SemaphoreType.DMA,
        ],
    )
    scatter_scratch = [
        pltpu.VMEM((ROWS_PER_TILE, CHW), jnp.int32),
        pltpu.VMEM((PER_TILE, HID), jnp.float32),
        pltpu.VMEM_SHARED((N, HID), jnp.float32),
        pltpu.SemaphoreType.DMA,
    ]
    sc_scatter = pl.kernel(
        _sc_scatter_body,
        out_type=jax.ShapeDtypeStruct((2, N, HID), jnp.float32),
        mesh=mesh,
        compiler_params=pltpu.CompilerParams(use_tc_tiling_on_sc=False),
        scratch_types=scatter_scratch,
    )
    sc_count = pl.kernel(
        _sc_count_body,
        out_type=jax.ShapeDtypeStruct((2, N, HID), jnp.float32),
        mesh=mesh,
        compiler_params=pltpu.CompilerParams(use_tc_tiling_on_sc=False),
        scratch_types=scatter_scratch,
    )
    return sc_gather, sc_scatter, sc_count


def _tc_msg_body(ef_ref, xs_ref, w1_ref, b1_ref, w2_ref, b2_ref, out_ref):
    i = pl.program_id(0)
    ef = ef_ref[...]
    hh = jnp.maximum(
        lax.dot_general(w1_ref[...], ef, (((1,), (1,)), ((), ())),
                        preferred_element_type=jnp.float32)
        + b1_ref[...], 0.0)
    wet = (lax.dot_general(w2_ref[...], hh.astype(jnp.bfloat16),
                           (((1,), (0,)), ((), ())),
                           preferred_element_type=jnp.float32)
           + b2_ref[...])
    xs_t = xs_ref[...].T
    wer = wet.reshape(HID, HID, B_EDGE)
    acc = xs_t[0:1] * wer[0]
    for h in range(1, HID):
        acc = acc + xs_t[h:h + 1] * wer[h]
    col = lax.broadcasted_iota(jnp.int32, (HID, B_EDGE), 1)
    acc = jnp.where(col < (E - i * B_EDGE), acc, 0.0)
    out_ref[...] = acc.T


_tc_msg = pl.pallas_call(
    _tc_msg_body,
    grid=(N_BLK,),
    in_specs=[
        pl.BlockSpec((B_EDGE, 5), lambda i: (i, 0)),
        pl.BlockSpec((B_EDGE, HID), lambda i: (i, 0)),
        pl.BlockSpec((IN_DIM, 5), lambda i: (0, 0)),
        pl.BlockSpec((IN_DIM, 1), lambda i: (0, 0)),
        pl.BlockSpec((HID * HID, IN_DIM), lambda i: (0, 0)),
        pl.BlockSpec((HID * HID, 1), lambda i: (0, 0)),
    ],
    out_specs=pl.BlockSpec((B_EDGE, HID), lambda i: (i, 0)),
    out_shape=jax.ShapeDtypeStruct((E_PAD, HID), jnp.float32),
)


def _tc_prep_body(nf_ref, w_ref, b_ref, out_ref):
    out_ref[...] = jnp.maximum(
        jnp.dot(nf_ref[...], w_ref[...], preferred_element_type=jnp.float32)
        + b_ref[...], 0.0)


_tc_prep = pl.pallas_call(
    _tc_prep_body,
    out_shape=jax.ShapeDtypeStruct((N, HID), jnp.float32),
)


def _tc_update_body(agg2_ref, cnt2_ref, x_ref, convb_ref,
                    wih_ref, whh_ref, bih_ref, bhh_ref, out_ref):
    agg = agg2_ref[0] + agg2_ref[1]
    cnt = cnt2_ref[0] + cnt2_ref[1]
    rowid = lax.broadcasted_iota(jnp.int32, (N, HID), 0)
    cnt = cnt - jnp.where(rowid == 0, float(E_PAD - E), 0.0)
    mean = agg / jnp.maximum(cnt, 1.0)
    m = jnp.maximum(mean + convb_ref[...], 0.0)
    x = x_ref[...]
    gi = jnp.dot(m, wih_ref[...], preferred_element_type=jnp.float32) + bih_ref[...]
    gh = jnp.dot(x, whh_ref[...], preferred_element_type=jnp.float32) + bhh_ref[...]
    r = jax.nn.sigmoid(gi[:, 0:HID] + gh[:, 0:HID])
    z = jax.nn.sigmoid(gi[:, HID:2 * HID] + gh[:, HID:2 * HID])
    n = jnp.tanh(gi[:, 2 * HID:] + r * gh[:, 2 * HID:])
    out_ref[...] = (1.0 - z) * n + z * x


_tc_update = pl.pallas_call(
    _tc_update_body,
    out_shape=jax.ShapeDtypeStruct((N, HID), jnp.float32),
)


def _tc_s2s_body(x_ref, wih_ref, whh_ref, bih_ref, bhh_ref, out_ref):
    xf = x_ref[...]
    q_star = jnp.zeros((1, 2 * HID), jnp.float32)
    h = jnp.zeros((1, HID), jnp.float32)
    c = jnp.zeros((1, HID), jnp.float32)
    for _ in range(3):
        g = (jnp.dot(q_star, wih_ref[...], preferred_element_type=jnp.float32)
             + bih_ref[...]
             + jnp.dot(h, whh_ref[...], preferred_element_type=jnp.float32)
             + bhh_ref[...])
        ii = jax.nn.sigmoid(g[:, 0:HID])
        f = jax.nn.sigmoid(g[:, HID:2 * HID])
        gg = jnp.tanh(g[:, 2 * HID:3 * HID])
        o = jax.nn.sigmoid(g[:, 3 * HID:])
        c = f * c + ii * gg
        h = o * jnp.tanh(c)
        e = jnp.sum(xf * h, axis=1, keepdims=True)
        e = e - jnp.max(e, axis=0, keepdims=True)
        a = jnp.exp(e)
        a = a / jnp.sum(a, axis=0, keepdims=True)
        r = jnp.sum(a * xf, axis=0, keepdims=True)
        q_star = jnp.concatenate([h, r], axis=1)
    out_ref[...] = q_star


_tc_s2s = pl.pallas_call(
    _tc_s2s_body,
    out_shape=jax.ShapeDtypeStruct((1, 2 * HID), jnp.float32),
)


def kernel(nfeat, efeat, lin0_w, lin0_b, blk1_w, blk1_b, blk2_w, blk2_b, conv_b,
           gru_wih, gru_whh, gru_bih, gru_bhh, s2s_wih, s2s_whh, s2s_bih,
           s2s_bhh, edge_index):
    f32 = jnp.float32
    pad_i = jnp.zeros((E_PAD - E,), jnp.int32)
    src2d = jnp.concatenate([edge_index[0], pad_i]).reshape(E_PAD // CHW, CHW)
    dst2d = jnp.concatenate([edge_index[1], pad_i]).reshape(E_PAD // CHW, CHW)
    efp = jnp.pad(efeat, ((0, E_PAD - E), (0, 0)))
    ones_t = jnp.ones((PER_TILE, HID), f32)
    zerosn = jnp.zeros((N, HID), f32)
    b1c = blk1_b[:, None]
    b2c = blk2_b[:, None]
    w2bf = blk2_w.astype(jnp.bfloat16)

    sc_gather, sc_scatter, sc_count = _sc_kernels()
    x = _tc_prep(nfeat, lin0_w.T, lin0_b[None])
    cnt2 = sc_count(ones_t, dst2d, zerosn)
    for _ in range(3):
        xs = sc_gather(x, src2d)
        msg = _tc_msg(efp, xs, blk1_w, b1c, w2bf, b2c)
        agg2 = sc_scatter(msg, dst2d, zerosn)
        x = _tc_update(agg2, cnt2, x, conv_b[None], gru_wih.T, gru_whh.T,
                       gru_bih[None], gru_bhh[None])
    q = _tc_s2s(x, s2s_wih.T, s2s_whh.T, s2s_bih[None], s2s_bhh[None])
    return (q, x)


# speedup vs baseline: 1.1237x; 1.1237x over previous
import functools

import jax
import jax.numpy as jnp
from jax import lax
from jax.experimental import pallas as pl
from jax.experimental.pallas import tpu as pltpu
from jax.experimental.pallas import tpu_sc as plsc

N = 10000
E = 160000
IN_DIM = 128
HID = 16

CHW = 128
E_PAD = 163840
ROWS_PER_TILE = 40
PER_TILE = ROWS_PER_TILE * CHW
N_PER_TILE = N // 16

B_EDGE = 4096
N_BLK = E_PAD // B_EDGE


def _sc_gather_body(x_hbm, src_hbm, out_hbm, idx_v, rows_v, sem):
    wid = lax.axis_index("s") * 2 + lax.axis_index("c")
    pltpu.sync_copy(src_hbm.at[pl.ds(wid * ROWS_PER_TILE, ROWS_PER_TILE)], idx_v)

    def chunk(i, carry):
        cps = []
        for j in range(8):
            k = i * 8 + j
            cps.append(pltpu.async_copy(
                x_hbm.at[idx_v.at[k]], rows_v.at[pl.ds(k * CHW, CHW)], sem))
        for cp in cps:
            cp.wait()
        return carry

    lax.fori_loop(0, ROWS_PER_TILE // 8, chunk, 0)
    pltpu.sync_copy(rows_v, out_hbm.at[pl.ds(wid * PER_TILE, PER_TILE)])


def _sc_scatter_body(vals_hbm, dst_hbm, zeros_hbm, out_hbm, idx_v, rows_v, agg_sh,
                     sem):
    cid = lax.axis_index("c")
    sid = lax.axis_index("s")
    wid = sid * 2 + cid
    nbase = sid * N_PER_TILE
    pltpu.sync_copy(zeros_hbm.at[pl.ds(nbase, N_PER_TILE)],
                    agg_sh.at[pl.ds(nbase, N_PER_TILE)])
    pltpu.sync_copy(dst_hbm.at[pl.ds(wid * ROWS_PER_TILE, ROWS_PER_TILE)], idx_v)
    pltpu.sync_copy(vals_hbm.at[pl.ds(wid * PER_TILE, PER_TILE)], rows_v)
    plsc.subcore_barrier()

    def chunk(i, carry):
        cps = []
        for j in range(8):
            k = i * 8 + j
            cps.append(pltpu.async_copy(rows_v.at[pl.ds(k * CHW, CHW)],
                                        agg_sh.at[idx_v.at[k]], sem, add=True))
        for cp in cps:
            cp.wait()
        return carry

    lax.fori_loop(0, ROWS_PER_TILE // 8, chunk, 0)
    plsc.subcore_barrier()
    pltpu.sync_copy(agg_sh.at[pl.ds(nbase, N_PER_TILE)],
                    out_hbm.at[cid, pl.ds(nbase, N_PER_TILE)])


def _sc_count_body(ones_hbm, dst_hbm, zeros_hbm, out_hbm, idx_v, rows_v, agg_sh,
                   sem):
    cid = lax.axis_index("c")
    sid = lax.axis_index("s")
    wid = sid * 2 + cid
    nbase = sid * N_PER_TILE
    pltpu.sync_copy(zeros_hbm.at[pl.ds(nbase, N_PER_TILE)],
                    agg_sh.at[pl.ds(nbase, N_PER_TILE)])
    pltpu.sync_copy(dst_hbm.at[pl.ds(wid * ROWS_PER_TILE, ROWS_PER_TILE)], idx_v)
    pltpu.sync_copy(ones_hbm, rows_v)
    plsc.subcore_barrier()

    def chunk(i, carry):
        cps = []
        for j in range(8):
            k = i * 8 + j
            cps.append(pltpu.async_copy(rows_v.at[pl.ds(k * CHW, CHW)],
                                        agg_sh.at[idx_v.at[k]], sem, add=True))
        for cp in cps:
            cp.wait()
        return carry

    lax.fori_loop(0, ROWS_PER_TILE // 8, chunk, 0)
    plsc.subcore_barrier()
    pltpu.sync_copy(agg_sh.at[pl.ds(nbase, N_PER_TILE)],
                    out_hbm.at[cid, pl.ds(nbase, N_PER_TILE)])


@functools.cache
def _sc_kernels():
    mesh = plsc.VectorSubcoreMesh(core_axis_name="c", subcore_axis_name="s")
    sc_gather = pl.kernel(
        _sc_gather_body,
        out_type=jax.ShapeDtypeStruct((E_PAD, HID), jnp.float32),
        mesh=mesh,
        compiler_params=pltpu.CompilerParams(use_tc_tiling_on_sc=False),
        scratch_types=[
            pltpu.VMEM((ROWS_PER_TILE, CHW), jnp.int32),
            pltpu.VMEM((PER_TILE, HID), jnp.float32),
            pltpu.SemaphoreType.DMA,
        ],
    )
    scatter_scratch = [
        pltpu.VMEM((ROWS_PER_TILE, CHW), jnp.int32),
        pltpu.VMEM((PER_TILE, HID), jnp.float32),
        pltpu.VMEM_SHARED((N, HID), jnp.float32),
        pltpu.SemaphoreType.DMA,
    ]
    sc_scatter = pl.kernel(
        _sc_scatter_body,
        out_type=jax.ShapeDtypeStruct((2, N, HID), jnp.float32),
        mesh=mesh,
        compiler_params=pltpu.CompilerParams(use_tc_tiling_on_sc=False),
        scratch_types=scatter_scratch,
    )
    sc_count = pl.kernel(
        _sc_count_body,
        out_type=jax.ShapeDtypeStruct((2, N, HID), jnp.float32),
        mesh=mesh,
        compiler_params=pltpu.CompilerParams(use_tc_tiling_on_sc=False),
        scratch_types=scatter_scratch,
    )
    return sc_gather, sc_scatter, sc_count


def _tc_msg_body(ef_ref, xs_ref, w1_ref, b1_ref, w2_ref, b2_ref, out_ref):
    i = pl.program_id(0)
    ef = ef_ref[...]
    hh = jnp.maximum(
        jnp.dot(w1_ref[...], ef, preferred_element_type=jnp.float32)
        + b1_ref[...], 0.0)
    wet = (lax.dot_general(w2_ref[...], hh.astype(jnp.bfloat16),
                           (((1,), (0,)), ((), ())),
                           preferred_element_type=jnp.float32)
           + b2_ref[...])
    xs_t = xs_ref[...].T
    wer = wet.reshape(HID, HID, B_EDGE)
    acc = xs_t[0:1] * wer[0]
    for h in range(1, HID):
        acc = acc + xs_t[h:h + 1] * wer[h]
    col = lax.broadcasted_iota(jnp.int32, (HID, B_EDGE), 1)
    acc = jnp.where(col < (E - i * B_EDGE), acc, 0.0)
    out_ref[...] = acc.T


_tc_msg = pl.pallas_call(
    _tc_msg_body,
    grid=(N_BLK,),
    in_specs=[
        pl.BlockSpec((5, B_EDGE), lambda i: (0, i)),
        pl.BlockSpec((B_EDGE, HID), lambda i: (i, 0)),
        pl.BlockSpec((IN_DIM, 5), lambda i: (0, 0)),
        pl.BlockSpec((IN_DIM, 1), lambda i: (0, 0)),
        pl.BlockSpec((HID * HID, IN_DIM), lambda i: (0, 0)),
        pl.BlockSpec((HID * HID, 1), lambda i: (0, 0)),
    ],
    out_specs=pl.BlockSpec((B_EDGE, HID), lambda i: (i, 0)),
    out_shape=jax.ShapeDtypeStruct((E_PAD, HID), jnp.float32),
)


def _tc_prep_body(nf_ref, w_ref, b_ref, out_ref):
    out_ref[...] = jnp.maximum(
        jnp.dot(nf_ref[...], w_ref[...], preferred_element_type=jnp.float32)
        + b_ref[...], 0.0)


_tc_prep = pl.pallas_call(
    _tc_prep_body,
    out_shape=jax.ShapeDtypeStruct((N, HID), jnp.float32),
)


def _tc_update_body(agg2_ref, cnt2_ref, x_ref, convb_ref,
                    wih_ref, whh_ref, bih_ref, bhh_ref, out_ref):
    agg = agg2_ref[0] + agg2_ref[1]
    cnt = cnt2_ref[0] + cnt2_ref[1]
    rowid = lax.broadcasted_iota(jnp.int32, (N, HID), 0)
    cnt = cnt - jnp.where(rowid == 0, float(E_PAD - E), 0.0)
    mean = agg / jnp.maximum(cnt, 1.0)
    m = jnp.maximum(mean + convb_ref[...], 0.0)
    x = x_ref[...]
    gi = jnp.dot(m, wih_ref[...], preferred_element_type=jnp.float32) + bih_ref[...]
    gh = jnp.dot(x, whh_ref[...], preferred_element_type=jnp.float32) + bhh_ref[...]
    r = jax.nn.sigmoid(gi[:, 0:HID] + gh[:, 0:HID])
    z = jax.nn.sigmoid(gi[:, HID:2 * HID] + gh[:, HID:2 * HID])
    n = jnp.tanh(gi[:, 2 * HID:] + r * gh[:, 2 * HID:])
    out_ref[...] = (1.0 - z) * n + z * x


_tc_update = pl.pallas_call(
    _tc_update_body,
    out_shape=jax.ShapeDtypeStruct((N, HID), jnp.float32),
)


def _tc_s2s_body(x_ref, wih_ref, whh_ref, bih_ref, bhh_ref, out_ref):
    xf = x_ref[...]
    q_star = jnp.zeros((1, 2 * HID), jnp.float32)
    h = jnp.zeros((1, HID), jnp.float32)
    c = jnp.zeros((1, HID), jnp.float32)
    for _ in range(3):
        g = (jnp.dot(q_star, wih_ref[...], preferred_element_type=jnp.float32)
             + bih_ref[...]
             + jnp.dot(h, whh_ref[...], preferred_element_type=jnp.float32)
             + bhh_ref[...])
        ii = jax.nn.sigmoid(g[:, 0:HID])
        f = jax.nn.sigmoid(g[:, HID:2 * HID])
        gg = jnp.tanh(g[:, 2 * HID:3 * HID])
        o = jax.nn.sigmoid(g[:, 3 * HID:])
        c = f * c + ii * gg
        h = o * jnp.tanh(c)
        e = jnp.sum(xf * h, axis=1, keepdims=True)
        e = e - jnp.max(e, axis=0, keepdims=True)
        a = jnp.exp(e)
        a = a / jnp.sum(a, axis=0, keepdims=True)
        r = jnp.sum(a * xf, axis=0, keepdims=True)
        q_star = jnp.concatenate([h, r], axis=1)
    out_ref[...] = q_star


_tc_s2s = pl.pallas_call(
    _tc_s2s_body,
    out_shape=jax.ShapeDtypeStruct((1, 2 * HID), jnp.float32),
)


def kernel(nfeat, efeat, lin0_w, lin0_b, blk1_w, blk1_b, blk2_w, blk2_b, conv_b,
           gru_wih, gru_whh, gru_bih, gru_bhh, s2s_wih, s2s_whh, s2s_bih,
           s2s_bhh, edge_index):
    f32 = jnp.float32
    pad_i = jnp.zeros((E_PAD - E,), jnp.int32)
    src2d = jnp.concatenate([edge_index[0], pad_i]).reshape(E_PAD // CHW, CHW)
    dst2d = jnp.concatenate([edge_index[1], pad_i]).reshape(E_PAD // CHW, CHW)
    eft = jnp.pad(efeat.T, ((0, 0), (0, E_PAD - E)))
    ones_t = jnp.ones((PER_TILE, HID), f32)
    zerosn = jnp.zeros((N, HID), f32)
    b1c = blk1_b[:, None]
    b2c = blk2_b[:, None]
    w2bf = blk2_w.astype(jnp.bfloat16)

    sc_gather, sc_scatter, sc_count = _sc_kernels()
    x = _tc_prep(nfeat, lin0_w.T, lin0_b[None])
    cnt2 = sc_count(ones_t, dst2d, zerosn)
    for _ in range(3):
        xs = sc_gather(x, src2d)
        msg = _tc_msg(eft, xs, blk1_w, b1c, w2bf, b2c)
        agg2 = sc_scatter(msg, dst2d, zerosn)
        x = _tc_update(agg2, cnt2, x, conv_b[None], gru_wih.T, gru_whh.T,
                       gru_bih[None], gru_bhh[None])
    q = _tc_s2s(x, s2s_wih.T, s2s_whh.T, s2s_bih[None], s2s_bhh[None])
    return (q, x)
